# baseline (speedup 1.0000x reference)
import functools

import jax
import jax.numpy as jnp
import numpy as np
from jax.experimental import pallas as pl
from jax.sharding import Mesh, PartitionSpec as P

_TC_BLOCK = 2048


def _tc_body(x_ref, y_ref, o_ref, *, C):
    x = x_ref[...]
    yv = y_ref[...]
    R = x.shape[0]
    col = jax.lax.broadcasted_iota(jnp.int32, (R, C), 1)
    onehot = col == yv[:, None]
    oy = jnp.sum(jnp.where(onehot, x, 0.0), axis=1, keepdims=True)
    hinge = jnp.maximum(x - oy + 1.0, 0.0)
    o_ref[...] = (jnp.sum(hinge, axis=1) - 1.0) * (1.0 / C)


def _local_kernel(output, y):
    B, C = output.shape
    R = min(_TC_BLOCK, B)
    return pl.pallas_call(
        functools.partial(_tc_body, C=C),
        grid=(B // R,),
        in_specs=[
            pl.BlockSpec((R, C), lambda i: (i, 0)),
            pl.BlockSpec((R,), lambda i: (i,)),
        ],
        out_specs=pl.BlockSpec((R,), lambda i: (i,)),
        out_shape=jax.ShapeDtypeStruct((B,), jnp.float32),
    )(output, y)


def kernel(output, y):
    devs = jax.devices()
    n = len(devs)
    if n == 1 or output.shape[0] % (n * _TC_BLOCK) != 0:
        return _local_kernel(output, y)
    mesh = Mesh(np.array(devs), ("b",))
    f = jax.shard_map(
        _local_kernel, mesh=mesh,
        in_specs=(P("b", None), P("b")), out_specs=P("b"),
        check_vma=False)
    return f(output, y)


# speedup vs baseline: 3.0387x; 3.0387x over previous
import functools

import jax
import jax.numpy as jnp
from jax import lax
from jax.experimental import pallas as pl
from jax.experimental.pallas import tpu as pltpu
from jax.experimental.pallas import tpu_sc as plsc

_NS = 16
_L = 16


def _sc_loss(x_hbm, y_hbm, o_hbm, y_v, buf, loss_v, sem0, sem1, *,
             C, row0, sc_rows):
    b_per_w = sc_rows // _NS
    n_chunks = b_per_w // _L
    wid = lax.axis_index("s")
    base_o = wid * b_per_w
    base_x = row0 + base_o
    lanes = lax.iota(jnp.int32, _L)
    n_full = C // _L
    rem = C % _L
    sems = (sem0, sem1)

    pltpu.sync_copy(y_hbm.at[pl.ds(base_x, b_per_w)], y_v)

    def start(g, b):
        pltpu.async_copy(x_hbm.at[pl.ds(base_x + g * _L, _L), :], buf.at[b], sems[b])

    start(0, 0)
    start(1, 1)

    def do_pair(p, _):
        for b in (0, 1):
            g = 2 * p + b
            pltpu.make_async_copy(
                x_hbm.at[pl.ds(0, _L), :], buf.at[b], sems[b]).wait()
            y16 = y_v[pl.ds(g * _L, _L)]

            def row(i, lvec):
                y_s = jnp.sum(jnp.where(lanes == i, y16, 0))
                ybc = jnp.full((_L,), y_s)
                cb = (y_s // _L) * _L
                vdiag = buf[b, i, pl.ds(cb, _L)]
                oy = jnp.sum(jnp.where(lanes + cb == ybc, vdiag, 0.0))
                av = jnp.full((_L,), oy - 1.0)
                acc = jnp.zeros((_L,), jnp.float32)
                for t in range(n_full - 1 if rem else n_full):
                    v = buf[b, i, pl.ds(t * _L, _L)]
                    acc = acc + jnp.maximum(v - av, 0.0)
                if rem:
                    v = buf[b, i, pl.ds((n_full - 1) * _L, _L)]
                    acc = acc + jnp.maximum(v - av, 0.0)
                    v = buf[b, i, pl.ds(C - _L, _L)]
                    r = jnp.maximum(v - av, 0.0)
                    r = jnp.where(lanes >= (_L - rem), r, 0.0)
                    acc = acc + r
                rowsum = jnp.sum(acc)
                return jnp.where(lanes == i, (rowsum - 1.0) * (1.0 / C), lvec)

            lvec = lax.fori_loop(0, _L, row, jnp.zeros((_L,), jnp.float32))
            loss_v[pl.ds(g * _L, _L)] = lvec

            @pl.when(g + 2 < n_chunks)
            def _():
                pltpu.async_copy(
                    x_hbm.at[pl.ds(base_x + (g + 2) * _L, _L), :],
                    buf.at[b], sems[b])

        return None

    lax.fori_loop(0, n_chunks // 2, do_pair, None)
    pltpu.sync_copy(loss_v, o_hbm.at[pl.ds(base_o, b_per_w)])


def _sc_call(output, y, row0, sc_rows):
    C = output.shape[1]
    b_per_w = sc_rows // _NS
    mesh = plsc.VectorSubcoreMesh(
        core_axis_name="c", subcore_axis_name="s", num_cores=1)
    return pl.kernel(
        functools.partial(_sc_loss, C=C, row0=row0, sc_rows=sc_rows),
        out_type=jax.ShapeDtypeStruct((sc_rows,), jnp.float32),
        mesh=mesh,
        compiler_params=pltpu.CompilerParams(needs_layout_passes=False),
        scratch_types=[
            pltpu.VMEM((b_per_w,), jnp.int32),
            pltpu.VMEM((2, _L, C), jnp.float32),
            pltpu.VMEM((b_per_w,), jnp.float32),
            pltpu.SemaphoreType.DMA,
            pltpu.SemaphoreType.DMA,
        ],
    )(output, y)


def kernel(output, y):
    B, C = output.shape
    half = B // 2
    p0 = _sc_call(output, y, 0, half)
    p1 = _sc_call(output, y, half, half)
    return jnp.concatenate([p0, p1])
